# baseline (speedup 1.0000x reference)
import functools

import jax
import jax.numpy as jnp
from jax import lax
from jax.experimental import pallas as pl
from jax.experimental.pallas import tpu as pltpu
from jax.experimental.pallas import tpu_sc as plsc

N = 10000
E = 160000
D = 256

NC = 2
NS = 16
NW = NC * NS
RW = 320
LASTV = N - RW * (NW - 1)
G = 128
SB = 1280
NB = E // SB
SUBS = SB // G
SENT = N

_sc_mesh = plsc.VectorSubcoreMesh(core_axis_name="c", subcore_axis_name="s")


@functools.partial(
    pl.kernel,
    out_type=jax.ShapeDtypeStruct((N, D), jnp.float32),
    mesh=_sc_mesh,
    scratch_types=[
        pltpu.VMEM((SB,), jnp.int32),
        pltpu.VMEM((SB,), jnp.int32),
        pltpu.VMEM((G,), jnp.int32),
        pltpu.VMEM((G, D), jnp.float32),
        pltpu.VMEM((RW, D), jnp.float32),
        pltpu.SemaphoreType.DMA,
    ],
    compiler_params=pltpu.CompilerParams(needs_layout_passes=False),
)
def _seg_kernel(x_hbm, src_hbm, dst_hbm, out_hbm, svm, dvm, gidx,
                rows, accv, sem):
    c = lax.axis_index("c")
    s = lax.axis_index("s")
    w = s * NC + c
    base = w * RW
    last = w == NW - 1

    @pl.when(jnp.logical_not(last))
    def _():
        pltpu.sync_copy(x_hbm.at[pl.ds(base, RW)], accv.at[pl.ds(0, RW)])

    @pl.when(last)
    def _():
        pltpu.sync_copy(x_hbm.at[pl.ds(base, LASTV)], accv.at[pl.ds(0, LASTV)])

    nv = jnp.where(last, LASTV, RW)
    hi = base + nv

    def blk(b, _):
        off = b * SB
        pltpu.sync_copy(src_hbm.at[pl.ds(off, SB)], svm)
        pltpu.sync_copy(dst_hbm.at[pl.ds(off, SB)], dvm)

        def sub(j, _):
            def rbody(k, _):
                sl = pl.ds(j * G + k * 16, 16)
                d = dvm[sl]
                ok = (d >= base) & (d < hi)
                gidx[pl.ds(k * 16, 16)] = jnp.where(ok, svm[sl], SENT)
                return 0

            lax.fori_loop(0, G // 16, rbody, 0)
            pltpu.async_copy(
                x_hbm.at[plsc.Indices(gidx, ignored_value=SENT)], rows, sem
            ).wait()

            def ebody(k, _):
                dv = dvm[pl.ds(j * G + k * 16, 16)] - base
                for i in range(16):
                    r = dv[i]

                    @pl.when((r >= 0) & (r < nv))
                    def _(r=r, k=k, i=i):
                        for cb in range(D // 16):
                            cs = pl.ds(cb * 16, 16)
                            plsc.addupdate(accv.at[r, cs],
                                           rows[k * 16 + i, cs])

                return 0

            lax.fori_loop(0, G // 16, ebody, 0)
            return 0

        lax.fori_loop(0, SUBS, sub, 0)
        return 0

    lax.fori_loop(0, NB, blk, 0)

    @pl.when(jnp.logical_not(last))
    def _():
        pltpu.sync_copy(accv.at[pl.ds(0, RW)], out_hbm.at[pl.ds(base, RW)])

    @pl.when(last)
    def _():
        pltpu.sync_copy(accv.at[pl.ds(0, LASTV)],
                        out_hbm.at[pl.ds(base, LASTV)])


BM = 1000


def _mlp_body(h_ref, wa_ref, ba_ref, wb_ref, bb_ref, o_ref):
    t = jnp.maximum(
        jnp.dot(h_ref[...], wa_ref[...], preferred_element_type=jnp.float32)
        + ba_ref[...], 0.0)
    o_ref[...] = jnp.maximum(
        jnp.dot(t, wb_ref[...], preferred_element_type=jnp.float32)
        + bb_ref[...], 0.0)


def _mlp_final_body(h_ref, wa_ref, ba_ref, wb_ref, bb_ref, wl_ref, bl_ref,
                    o_ref):
    t = jnp.maximum(
        jnp.dot(h_ref[...], wa_ref[...], preferred_element_type=jnp.float32)
        + ba_ref[...], 0.0)
    u = jnp.maximum(
        jnp.dot(t, wb_ref[...], preferred_element_type=jnp.float32)
        + bb_ref[...], 0.0)
    o_ref[...] = (jnp.dot(u, wl_ref[...], preferred_element_type=jnp.float32)
                  + bl_ref[...])


_row_spec = pl.BlockSpec((BM, D), lambda i: (i, 0))
_mat_spec = pl.BlockSpec((D, D), lambda i: (0, 0))
_bias_spec = pl.BlockSpec((1, D), lambda i: (0, 0))


def _mlp(h, wa, ba, wb, bb):
    return pl.pallas_call(
        _mlp_body,
        grid=(N // BM,),
        in_specs=[_row_spec, _mat_spec, _bias_spec, _mat_spec, _bias_spec],
        out_specs=_row_spec,
        out_shape=jax.ShapeDtypeStruct((N, D), jnp.float32),
    )(h, wa, ba.reshape(1, D), wb, bb.reshape(1, D))


def _mlp_final(h, wa, ba, wb, bb, wl, bl):
    return pl.pallas_call(
        _mlp_final_body,
        grid=(N // BM,),
        in_specs=[_row_spec, _mat_spec, _bias_spec, _mat_spec, _bias_spec,
                  _mat_spec, _bias_spec],
        out_specs=_row_spec,
        out_shape=jax.ShapeDtypeStruct((N, D), jnp.float32),
    )(h, wa, ba.reshape(1, D), wb, bb.reshape(1, D), wl, bl.reshape(1, D))


def kernel(x, edge_index, W1a, b1a, W1b, b1b, W2a, b2a, W2b, b2b, Wl, bl):
    ei = edge_index.astype(jnp.int32)
    src, dst = ei[0], ei[1]
    s1 = _seg_kernel(x, src, dst)
    h1 = _mlp(s1, W1a, b1a, W1b, b1b)
    s2 = _seg_kernel(h1, src, dst)
    return _mlp_final(s2, W2a, b2a, W2b, b2b, Wl, bl)
